# baseline (speedup 1.0000x reference)
import functools

import jax
import jax.numpy as jnp
from jax import lax
from jax.experimental import pallas as pl
from jax.experimental.pallas import tpu as pltpu
from jax.experimental.pallas import tpu_sc as plsc

_N = 8192
_K = 64
_SPLIT = 5120

_BLOCK_ROWS = 128

_L = 16
_NW = 32
_SC_ROWS = _N - _SPLIT
_ROWS_PER_W = _SC_ROWS // _NW
_ROW_VREGS = _N // _L
_CAND = _N + _L


def _tc_body(w_ref, a_ref, m_ref, o_ref):
    m = m_ref[...]
    relu_w = jnp.maximum(w_ref[...], 0.0)
    adj = relu_w + m * (a_ref[...] - relu_w)
    bits = jax.lax.bitcast_convert_type(adj, jnp.int32)

    gm = bits[:, 0:64]
    for k in range(1, 128):
        gm = jnp.maximum(gm, bits[:, k * 64:(k + 1) * 64])
    lo = jnp.min(gm, axis=1, keepdims=True)
    hi = jnp.max(gm, axis=1, keepdims=True) + 1

    def cond(state):
        lo_, hi_ = state
        return jnp.any(hi_ - lo_ > 1)

    def body(state):
        lo_, hi_ = state
        mid = lo_ + ((hi_ - lo_) >> 1)
        cnt = jnp.sum((bits >= mid).astype(jnp.int32), axis=1, keepdims=True)
        ge = cnt >= _K
        return jnp.where(ge, mid, lo_), jnp.where(ge, hi_, mid)

    lo, hi = jax.lax.while_loop(cond, body, (lo, hi))
    o_ref[...] = jnp.where(bits >= lo, adj, 0.0)


def _sc_body(w_hbm, a_hbm, m_hbm, o_hbm,
             wv0, av0, mv0, adj0, wv1, av1, mv1, adj1, candv,
             sin0, sin1, sout0, sout1):
    c = lax.axis_index("c")
    s = lax.axis_index("s")
    wid = s * 2 + c
    row0 = _SPLIT + wid * _ROWS_PER_W
    slots = ((wv0, av0, mv0, adj0, sin0, sout0),
             (wv1, av1, mv1, adj1, sin1, sout1))

    def issue_in(row, slot):
        wv, av, mv, _, sin, _ = slots[slot]
        pltpu.async_copy(w_hbm.at[row], wv, sin)
        pltpu.async_copy(a_hbm.at[row], av, sin)
        pltpu.async_copy(m_hbm.at[row], mv, sin)

    def wait_in(slot):
        wv, av, mv, _, sin, _ = slots[slot]
        pltpu.make_async_copy(w_hbm.at[row0], wv, sin).wait()
        pltpu.make_async_copy(a_hbm.at[row0], av, sin).wait()
        pltpu.make_async_copy(m_hbm.at[row0], mv, sin).wait()

    def issue_out(row, slot):
        _, _, _, adjv, _, sout = slots[slot]
        pltpu.async_copy(adjv, o_hbm.at[row - _SPLIT], sout)

    def wait_out(slot):
        _, _, _, adjv, _, sout = slots[slot]
        pltpu.make_async_copy(adjv, o_hbm.at[0], sout).wait()

    def process_row(row, slot):
        wv, av, mv, adjv, _, _ = slots[slot]
        neg = jnp.full((_L,), jnp.int32(-1))

        def cpass(i, gms):
            gm0, gm1, gm2, gm3 = gms
            new = []
            for u in range(4):
                j = i * 4 + u
                w = wv[pl.ds(j * _L, _L)]
                a = av[pl.ds(j * _L, _L)]
                m = mv[pl.ds(j * _L, _L)]
                relu = jnp.maximum(w, 0.0)
                adj = relu + m * (a - relu)
                adjv[pl.ds(j * _L, _L)] = adj
                bits = lax.bitcast_convert_type(adj, jnp.int32)
                new.append(jnp.maximum((gm0, gm1, gm2, gm3)[u], bits))
            return tuple(new)

        gm0, gm1, gm2, gm3 = lax.fori_loop(
            0, _ROW_VREGS // 4, cpass, (neg, neg, neg, neg))
        gmin = jnp.minimum(jnp.minimum(gm0, gm1), jnp.minimum(gm2, gm3))
        gmax = jnp.maximum(jnp.maximum(gm0, gm1), jnp.maximum(gm2, gm3))
        lo = jnp.min(gmin)
        hi = jnp.max(gmax) + 1

        def comp(i, off):
            v = adjv[pl.ds(i * _L, _L)]
            bits = lax.bitcast_convert_type(v, jnp.int32)
            msk = bits >= lo
            plsc.store_compressed(candv.at[pl.ds(off, _L)], bits, mask=msk)
            cnt = plsc.all_reduce_population_count(msk)
            return off + cnt[0]

        cnt = lax.fori_loop(0, _ROW_VREGS, comp, jnp.int32(0))
        nv = (cnt + _L - 1) // _L
        lane = lax.iota(jnp.int32, _L)

        def bis_cond(st):
            return st[1] - st[0] > 1

        def bis_body(st):
            lo_, hi_ = st
            mid = lo_ + ((hi_ - lo_) >> 1)

            def cc(i, acc):
                v = candv[pl.ds(i * _L, _L)]
                valid = (lane + i * _L) < cnt
                return acc + jnp.where((v >= mid) & valid, 1, 0)

            accv = lax.fori_loop(0, nv, cc, jnp.zeros((_L,), jnp.int32))
            ge = jnp.sum(accv) >= _K
            return (jnp.where(ge, mid, lo_), jnp.where(ge, hi_, mid))

        t, _ = lax.while_loop(bis_cond, bis_body, (lo, hi))

        def mpass(i, _):
            v = adjv[pl.ds(i * _L, _L)]
            bits = lax.bitcast_convert_type(v, jnp.int32)
            adjv[pl.ds(i * _L, _L)] = jnp.where(bits >= t, v, 0.0)
            return 0

        lax.fori_loop(0, _ROW_VREGS, mpass, 0)

    issue_in(row0, 0)

    def pair_body(g, carry):
        r = g * 2
        issue_in(row0 + r + 1, 1)
        wait_in(0)

        @pl.when(g > 0)
        def _():
            wait_out(0)

        process_row(row0 + r, 0)
        issue_out(row0 + r, 0)

        @pl.when(r + 2 < _ROWS_PER_W)
        def _():
            issue_in(row0 + r + 2, 0)

        wait_in(1)

        @pl.when(g > 0)
        def _():
            wait_out(1)

        process_row(row0 + r + 1, 1)
        issue_out(row0 + r + 1, 1)
        return carry

    lax.fori_loop(0, _ROWS_PER_W // 2, pair_body, 0)
    wait_out(0)
    wait_out(1)


@jax.jit
def kernel(idx, W, A_prior, freeze_mask):
    del idx
    in_spec = pl.BlockSpec((_BLOCK_ROWS, _N), lambda i: (i, 0))
    tc_out = pl.pallas_call(
        _tc_body,
        grid=(_SPLIT // _BLOCK_ROWS,),
        in_specs=[in_spec, in_spec, in_spec],
        out_specs=pl.BlockSpec((_BLOCK_ROWS, _N), lambda i: (i, 0)),
        out_shape=jax.ShapeDtypeStruct((_N, _N), jnp.float32),
    )(W, A_prior, freeze_mask)

    mesh = plsc.VectorSubcoreMesh(core_axis_name="c", subcore_axis_name="s")
    sc_out = pl.kernel(
        _sc_body,
        out_type=jax.ShapeDtypeStruct((_SC_ROWS, _N), jnp.float32),
        mesh=mesh,
        compiler_params=pltpu.CompilerParams(needs_layout_passes=False),
        scratch_types=(
            [pltpu.VMEM((_N,), jnp.float32)] * 8
            + [pltpu.VMEM((_CAND,), jnp.int32)]
            + [pltpu.SemaphoreType.DMA] * 4
        ),
    )(W, A_prior, freeze_mask)
    return lax.dynamic_update_slice(tc_out, sc_out, (_SPLIT, 0))


# speedup vs baseline: 1.1500x; 1.1500x over previous
import functools

import jax
import jax.numpy as jnp
from jax import lax
from jax.experimental import pallas as pl
from jax.experimental.pallas import tpu as pltpu
from jax.experimental.pallas import tpu_sc as plsc

_N = 8192
_K = 64
_SPLIT = 5120

_BLOCK_ROWS = 128

_L = 16
_NW = 32
_SC_ROWS = _N - _SPLIT
_ROWS_PER_W = _SC_ROWS // _NW
_ROW_VREGS = _N // _L
_CAND = _N + _L


def _tc_body(w_ref, a_ref, m_ref, o_ref):
    m = m_ref[...]
    relu_w = jnp.maximum(w_ref[...], 0.0)
    adj = relu_w + m * (a_ref[...] - relu_w)
    bits = jax.lax.bitcast_convert_type(adj, jnp.int32)

    gm = bits[:, 0:64]
    for k in range(1, 128):
        gm = jnp.maximum(gm, bits[:, k * 64:(k + 1) * 64])
    lo = jnp.min(gm, axis=1, keepdims=True)
    hi = jnp.max(gm, axis=1, keepdims=True) + 1

    def cond(state):
        lo_, hi_ = state
        return jnp.any(hi_ - lo_ > 1)

    def body(state):
        lo_, hi_ = state
        mid = lo_ + ((hi_ - lo_) >> 1)
        cnt = jnp.sum((bits >= mid).astype(jnp.int32), axis=1, keepdims=True)
        ge = cnt >= _K
        return jnp.where(ge, mid, lo_), jnp.where(ge, hi_, mid)

    lo, hi = jax.lax.while_loop(cond, body, (lo, hi))
    o_ref[...] = jnp.where(bits >= lo, adj, 0.0)


def _sc_body(w_hbm, a_hbm, m_hbm, o_hbm,
             wv0, av0, mv0, adj0, wv1, av1, mv1, adj1, candv,
             sin0, sin1, sout0, sout1):
    c = lax.axis_index("c")
    s = lax.axis_index("s")
    wid = s * 2 + c
    row0 = _SPLIT + wid * _ROWS_PER_W
    slots = ((wv0, av0, mv0, adj0, sin0, sout0),
             (wv1, av1, mv1, adj1, sin1, sout1))

    def issue_in(row, slot):
        wv, av, mv, _, sin, _ = slots[slot]
        pltpu.async_copy(w_hbm.at[row], wv, sin)
        pltpu.async_copy(a_hbm.at[row], av, sin)
        pltpu.async_copy(m_hbm.at[row], mv, sin)

    def wait_in(slot):
        wv, av, mv, _, sin, _ = slots[slot]
        pltpu.make_async_copy(w_hbm.at[row0], wv, sin).wait()
        pltpu.make_async_copy(a_hbm.at[row0], av, sin).wait()
        pltpu.make_async_copy(m_hbm.at[row0], mv, sin).wait()

    def issue_out(row, slot):
        _, _, _, adjv, _, sout = slots[slot]
        pltpu.async_copy(adjv, o_hbm.at[row - _SPLIT], sout)

    def wait_out(slot):
        _, _, _, adjv, _, sout = slots[slot]
        pltpu.make_async_copy(adjv, o_hbm.at[0], sout).wait()

    def process_row(row, slot):
        wv, av, mv, adjv, _, _ = slots[slot]
        neg = jnp.full((_L,), jnp.int32(-1))

        def cpass(i, gms):
            gm0, gm1, gm2, gm3 = gms
            new = []
            for u in range(4):
                j = i * 4 + u
                w = wv[pl.ds(j * _L, _L)]
                a = av[pl.ds(j * _L, _L)]
                m = mv[pl.ds(j * _L, _L)]
                relu = jnp.maximum(w, 0.0)
                adj = relu + m * (a - relu)
                adjv[pl.ds(j * _L, _L)] = adj
                bits = lax.bitcast_convert_type(adj, jnp.int32)
                new.append(jnp.maximum((gm0, gm1, gm2, gm3)[u], bits))
            return tuple(new)

        gm0, gm1, gm2, gm3 = lax.fori_loop(
            0, _ROW_VREGS // 4, cpass, (neg, neg, neg, neg))
        gmin = jnp.minimum(jnp.minimum(gm0, gm1), jnp.minimum(gm2, gm3))
        gmax = jnp.maximum(jnp.maximum(gm0, gm1), jnp.maximum(gm2, gm3))
        lo = jnp.min(gmin)
        hi = jnp.max(gmax) + 1

        def comp(i, offv):
            for u in range(4):
                j = i * 4 + u
                v = adjv[pl.ds(j * _L, _L)]
                bits = lax.bitcast_convert_type(v, jnp.int32)
                msk = bits >= lo
                mint = msk.astype(jnp.int32)
                idx = offv + plsc.cumsum(mint) - mint
                plsc.store_scatter(candv, (idx,), bits, mask=msk)
                offv = offv + plsc.all_reduce_population_count(msk)
            return offv

        offv = lax.fori_loop(0, _ROW_VREGS // 4, comp,
                             jnp.zeros((_L,), jnp.int32))
        cnt = offv[0]
        nv = (cnt + _L - 1) // _L
        lane = lax.iota(jnp.int32, _L)

        def bis_cond(st):
            return st[1] - st[0] > 1

        def bis_body(st):
            lo_, hi_ = st
            mid = lo_ + ((hi_ - lo_) >> 1)

            def cc(i, acc):
                v = candv[pl.ds(i * _L, _L)]
                valid = (lane + i * _L) < cnt
                return acc + jnp.where((v >= mid) & valid, 1, 0)

            accv = lax.fori_loop(0, nv, cc, jnp.zeros((_L,), jnp.int32))
            ge = jnp.sum(accv) >= _K
            return (jnp.where(ge, mid, lo_), jnp.where(ge, hi_, mid))

        t, _ = lax.while_loop(bis_cond, bis_body, (lo, hi))

        def mpass(i, _):
            for u in range(4):
                j = i * 4 + u
                v = adjv[pl.ds(j * _L, _L)]
                bits = lax.bitcast_convert_type(v, jnp.int32)
                adjv[pl.ds(j * _L, _L)] = jnp.where(bits >= t, v, 0.0)
            return 0

        lax.fori_loop(0, _ROW_VREGS // 4, mpass, 0)

    issue_in(row0, 0)

    def pair_body(g, carry):
        r = g * 2
        issue_in(row0 + r + 1, 1)
        wait_in(0)

        @pl.when(g > 0)
        def _():
            wait_out(0)

        process_row(row0 + r, 0)
        issue_out(row0 + r, 0)

        @pl.when(r + 2 < _ROWS_PER_W)
        def _():
            issue_in(row0 + r + 2, 0)

        wait_in(1)

        @pl.when(g > 0)
        def _():
            wait_out(1)

        process_row(row0 + r + 1, 1)
        issue_out(row0 + r + 1, 1)
        return carry

    lax.fori_loop(0, _ROWS_PER_W // 2, pair_body, 0)
    wait_out(0)
    wait_out(1)


@jax.jit
def kernel(idx, W, A_prior, freeze_mask):
    del idx
    in_spec = pl.BlockSpec((_BLOCK_ROWS, _N), lambda i: (i, 0))
    tc_out = pl.pallas_call(
        _tc_body,
        grid=(_SPLIT // _BLOCK_ROWS,),
        in_specs=[in_spec, in_spec, in_spec],
        out_specs=pl.BlockSpec((_BLOCK_ROWS, _N), lambda i: (i, 0)),
        out_shape=jax.ShapeDtypeStruct((_N, _N), jnp.float32),
    )(W, A_prior, freeze_mask)

    mesh = plsc.VectorSubcoreMesh(core_axis_name="c", subcore_axis_name="s")
    sc_out = pl.kernel(
        _sc_body,
        out_type=jax.ShapeDtypeStruct((_SC_ROWS, _N), jnp.float32),
        mesh=mesh,
        compiler_params=pltpu.CompilerParams(needs_layout_passes=False),
        scratch_types=(
            [pltpu.VMEM((_N,), jnp.float32)] * 8
            + [pltpu.VMEM((_CAND,), jnp.int32)]
            + [pltpu.SemaphoreType.DMA] * 4
        ),
    )(W, A_prior, freeze_mask)
    return lax.dynamic_update_slice(tc_out, sc_out, (_SPLIT, 0))


# speedup vs baseline: 1.4119x; 1.2278x over previous
import functools

import jax
import jax.numpy as jnp
from jax import lax
from jax.experimental import pallas as pl
from jax.experimental.pallas import tpu as pltpu
from jax.experimental.pallas import tpu_sc as plsc

_N = 8192
_K = 64
_SPLIT = 5760

_BLOCK_ROWS = 128

_L = 16
_NW = 32
_SC_ROWS = _N - _SPLIT
_ROWS_PER_W = _SC_ROWS // _NW
_ROW_VREGS = _N // _L
_CAND = _N + _L


def _tc_body(w_ref, a_ref, m_ref, o_ref):
    m = m_ref[...]
    relu_w = jnp.maximum(w_ref[...], 0.0)
    adj = relu_w + m * (a_ref[...] - relu_w)
    bits = jax.lax.bitcast_convert_type(adj, jnp.int32)

    gm = bits[:, 0:64]
    for k in range(1, 128):
        gm = jnp.maximum(gm, bits[:, k * 64:(k + 1) * 64])
    lo = jnp.min(gm, axis=1, keepdims=True)
    hi = jnp.max(gm, axis=1, keepdims=True) + 1

    def cond(state):
        lo_, hi_ = state
        return jnp.any(hi_ - lo_ > 1)

    def body(state):
        lo_, hi_ = state
        mid = lo_ + ((hi_ - lo_) >> 1)
        cnt = jnp.sum((bits >= mid).astype(jnp.int32), axis=1, keepdims=True)
        ge = cnt >= _K
        return jnp.where(ge, mid, lo_), jnp.where(ge, hi_, mid)

    lo, hi = jax.lax.while_loop(cond, body, (lo, hi))
    o_ref[...] = jnp.where(bits >= lo, adj, 0.0)


def _sc_body(w_hbm, a_hbm, m_hbm, o_hbm,
             wv0, av0, mv0, adj0, wv1, av1, mv1, adj1, candv,
             sin0, sin1, sout0, sout1):
    c = lax.axis_index("c")
    s = lax.axis_index("s")
    wid = s * 2 + c
    row0 = _SPLIT + wid * _ROWS_PER_W
    slots = ((wv0, av0, mv0, adj0, sin0, sout0),
             (wv1, av1, mv1, adj1, sin1, sout1))

    def issue_in(row, slot):
        wv, av, mv, _, sin, _ = slots[slot]
        pltpu.async_copy(w_hbm.at[row], wv, sin)
        pltpu.async_copy(a_hbm.at[row], av, sin)
        pltpu.async_copy(m_hbm.at[row], mv, sin)

    def wait_in(slot):
        wv, av, mv, _, sin, _ = slots[slot]
        pltpu.make_async_copy(w_hbm.at[row0], wv, sin).wait()
        pltpu.make_async_copy(a_hbm.at[row0], av, sin).wait()
        pltpu.make_async_copy(m_hbm.at[row0], mv, sin).wait()

    def issue_out(row, slot):
        _, _, _, adjv, _, sout = slots[slot]
        pltpu.async_copy(adjv, o_hbm.at[row - _SPLIT], sout)

    def wait_out(slot):
        _, _, _, adjv, _, sout = slots[slot]
        pltpu.make_async_copy(adjv, o_hbm.at[0], sout).wait()

    def process_row(row, slot):
        wv, av, mv, adjv, _, _ = slots[slot]
        neg = jnp.full((_L,), jnp.int32(-1))

        def cpass(i, gms):
            gm0, gm1, gm2, gm3 = gms
            new = []
            for u in range(4):
                j = i * 4 + u
                w = wv[pl.ds(j * _L, _L)]
                a = av[pl.ds(j * _L, _L)]
                m = mv[pl.ds(j * _L, _L)]
                relu = jnp.maximum(w, 0.0)
                adj = relu + m * (a - relu)
                adjv[pl.ds(j * _L, _L)] = adj
                bits = lax.bitcast_convert_type(adj, jnp.int32)
                new.append(jnp.maximum((gm0, gm1, gm2, gm3)[u], bits))
            return tuple(new)

        gm0, gm1, gm2, gm3 = lax.fori_loop(
            0, _ROW_VREGS // 4, cpass, (neg, neg, neg, neg))
        gmin = jnp.minimum(jnp.minimum(gm0, gm1), jnp.minimum(gm2, gm3))
        gmax = jnp.maximum(jnp.maximum(gm0, gm1), jnp.maximum(gm2, gm3))
        lo = jnp.min(gmin)
        hi = jnp.max(gmax) + 1

        def comp(i, offv):
            for u in range(4):
                j = i * 4 + u
                v = adjv[pl.ds(j * _L, _L)]
                bits = lax.bitcast_convert_type(v, jnp.int32)
                msk = bits >= lo
                mint = msk.astype(jnp.int32)
                idx = offv + plsc.cumsum(mint) - mint
                plsc.store_scatter(candv, (idx,), bits, mask=msk)
                offv = offv + plsc.all_reduce_population_count(msk)
            return offv

        offv = lax.fori_loop(0, _ROW_VREGS // 4, comp,
                             jnp.zeros((_L,), jnp.int32))
        cnt = offv[0]
        nv = (cnt + _L - 1) // _L
        lane = lax.iota(jnp.int32, _L)

        def bis_cond(st):
            return st[1] - st[0] > 1

        def bis_body(st):
            lo_, hi_ = st
            mid = lo_ + ((hi_ - lo_) >> 1)

            def cc(i, acc):
                v = candv[pl.ds(i * _L, _L)]
                valid = (lane + i * _L) < cnt
                return acc + jnp.where((v >= mid) & valid, 1, 0)

            accv = lax.fori_loop(0, nv, cc, jnp.zeros((_L,), jnp.int32))
            ge = jnp.sum(accv) >= _K
            return (jnp.where(ge, mid, lo_), jnp.where(ge, hi_, mid))

        t, _ = lax.while_loop(bis_cond, bis_body, (lo, hi))

        def mpass(i, _):
            for u in range(4):
                j = i * 4 + u
                v = adjv[pl.ds(j * _L, _L)]
                bits = lax.bitcast_convert_type(v, jnp.int32)
                adjv[pl.ds(j * _L, _L)] = jnp.where(bits >= t, v, 0.0)
            return 0

        lax.fori_loop(0, _ROW_VREGS // 4, mpass, 0)

    issue_in(row0, 0)

    def pair_body(g, carry):
        r = g * 2
        issue_in(row0 + r + 1, 1)
        wait_in(0)

        @pl.when(g > 0)
        def _():
            wait_out(0)

        process_row(row0 + r, 0)
        issue_out(row0 + r, 0)

        @pl.when(r + 2 < _ROWS_PER_W)
        def _():
            issue_in(row0 + r + 2, 0)

        wait_in(1)

        @pl.when(g > 0)
        def _():
            wait_out(1)

        process_row(row0 + r + 1, 1)
        issue_out(row0 + r + 1, 1)
        return carry

    lax.fori_loop(0, _ROWS_PER_W // 2, pair_body, 0)
    wait_out(0)
    wait_out(1)


@jax.jit
def kernel(idx, W, A_prior, freeze_mask):
    del idx
    in_spec = pl.BlockSpec((_BLOCK_ROWS, _N), lambda i: (i, 0))
    tc_out = pl.pallas_call(
        _tc_body,
        grid=(_SPLIT // _BLOCK_ROWS,),
        in_specs=[in_spec, in_spec, in_spec],
        out_specs=pl.BlockSpec((_BLOCK_ROWS, _N), lambda i: (i, 0)),
        out_shape=jax.ShapeDtypeStruct((_N, _N), jnp.float32),
    )(W, A_prior, freeze_mask)

    mesh = plsc.VectorSubcoreMesh(core_axis_name="c", subcore_axis_name="s")
    sc_out = pl.kernel(
        _sc_body,
        out_type=jax.ShapeDtypeStruct((_SC_ROWS, _N), jnp.float32),
        mesh=mesh,
        compiler_params=pltpu.CompilerParams(needs_layout_passes=False),
        scratch_types=(
            [pltpu.VMEM((_N,), jnp.float32)] * 8
            + [pltpu.VMEM((_CAND,), jnp.int32)]
            + [pltpu.SemaphoreType.DMA] * 4
        ),
    )(W, A_prior, freeze_mask)
    return lax.dynamic_update_slice(tc_out, sc_out, (_SPLIT, 0))


# speedup vs baseline: 1.4123x; 1.0003x over previous
import jax
import jax.numpy as jnp
from jax import lax
from jax.experimental import pallas as pl
from jax.experimental.pallas import tpu as pltpu
from jax.experimental.pallas import tpu_sc as plsc

_N = 8192
_K = 64
_SPLIT = 5760

_BLOCK_ROWS = 128

_L = 16
_NW = 32
_SC_ROWS = _N - _SPLIT
_ROWS_PER_W = _SC_ROWS // _NW
_ROW_VREGS = _N // _L
_CAND = _N + _L


def _tc_body(w_ref, a_ref, m_ref, o_ref):
    m = m_ref[...]
    relu_w = jnp.maximum(w_ref[...], 0.0)
    adj = relu_w + m * (a_ref[...] - relu_w)
    bits = jax.lax.bitcast_convert_type(adj, jnp.int32)

    gm = bits[:, 0:64]
    for k in range(1, 128):
        gm = jnp.maximum(gm, bits[:, k * 64:(k + 1) * 64])
    lo = jnp.min(gm, axis=1, keepdims=True)
    hi = jnp.max(gm, axis=1, keepdims=True) + 1

    def cond(state):
        lo_, hi_ = state
        return jnp.any(hi_ - lo_ > 1)

    def body(state):
        lo_, hi_ = state
        mid = lo_ + ((hi_ - lo_) >> 1)
        cnt = jnp.sum((bits >= mid).astype(jnp.int32), axis=1, keepdims=True)
        ge = cnt >= _K
        return jnp.where(ge, mid, lo_), jnp.where(ge, hi_, mid)

    lo, hi = jax.lax.while_loop(cond, body, (lo, hi))
    o_ref[...] = jnp.where(bits >= lo, adj, 0.0)


def _sc_body(w_hbm, a_hbm, m_hbm, o_hbm,
             wv0, av0, mv0, adj0, wv1, av1, mv1, adj1, candv,
             sin0, sin1, sout0, sout1):
    c = lax.axis_index("c")
    s = lax.axis_index("s")
    wid = s * 2 + c
    row0 = _SPLIT + wid * _ROWS_PER_W
    slots = ((wv0, av0, mv0, adj0, sin0, sout0),
             (wv1, av1, mv1, adj1, sin1, sout1))

    def issue_in(row, slot):
        wv, av, mv, _, sin, _ = slots[slot]
        pltpu.async_copy(w_hbm.at[row], wv, sin)
        pltpu.async_copy(a_hbm.at[row], av, sin)
        pltpu.async_copy(m_hbm.at[row], mv, sin)

    def wait_in(slot):
        wv, av, mv, _, sin, _ = slots[slot]
        pltpu.make_async_copy(w_hbm.at[row0], wv, sin).wait()
        pltpu.make_async_copy(a_hbm.at[row0], av, sin).wait()
        pltpu.make_async_copy(m_hbm.at[row0], mv, sin).wait()

    def issue_out(row, slot):
        _, _, _, adjv, _, sout = slots[slot]
        pltpu.async_copy(adjv, o_hbm.at[row - _SPLIT], sout)

    def wait_out(slot):
        _, _, _, adjv, _, sout = slots[slot]
        pltpu.make_async_copy(adjv, o_hbm.at[0], sout).wait()

    def process_row(row, slot):
        wv, av, mv, adjv, _, _ = slots[slot]
        neg = jnp.full((_L,), jnp.int32(-1))

        @plsc.parallel_loop(0, _ROW_VREGS, 4, unroll=4,
                           carry=(neg, neg, neg, neg))
        def cpass(j4, gms):
            gm0, gm1, gm2, gm3 = gms
            new = []
            for u in range(4):
                j = j4 + u
                w = wv[pl.ds(j * _L, _L)]
                a = av[pl.ds(j * _L, _L)]
                m = mv[pl.ds(j * _L, _L)]
                relu = jnp.maximum(w, 0.0)
                adj = relu + m * (a - relu)
                adjv[pl.ds(j * _L, _L)] = adj
                bits = lax.bitcast_convert_type(adj, jnp.int32)
                new.append(jnp.maximum((gm0, gm1, gm2, gm3)[u], bits))
            return tuple(new)

        gm0, gm1, gm2, gm3 = cpass
        gmin = jnp.minimum(jnp.minimum(gm0, gm1), jnp.minimum(gm2, gm3))
        gmax = jnp.maximum(jnp.maximum(gm0, gm1), jnp.maximum(gm2, gm3))
        lo = jnp.min(gmin)
        hi = jnp.max(gmax) + 1

        @plsc.parallel_loop(0, _ROW_VREGS, 4, unroll=4,
                           carry=jnp.zeros((_L,), jnp.int32))
        def comp(j4, offv):
            for u in range(4):
                j = j4 + u
                v = adjv[pl.ds(j * _L, _L)]
                bits = lax.bitcast_convert_type(v, jnp.int32)
                msk = bits >= lo
                mint = msk.astype(jnp.int32)
                idx = offv + plsc.cumsum(mint) - mint
                plsc.store_scatter(candv, (idx,), bits, mask=msk)
                offv = offv + plsc.all_reduce_population_count(msk)
            return offv

        cnt = comp[0]
        nv = (cnt + _L - 1) // _L
        lane = lax.iota(jnp.int32, _L)

        def bis_cond(st):
            return st[1] - st[0] > 1

        def bis_body(st):
            lo_, hi_ = st
            mid = lo_ + ((hi_ - lo_) >> 1)

            def cc(i, acc):
                v = candv[pl.ds(i * _L, _L)]
                valid = (lane + i * _L) < cnt
                return acc + jnp.where((v >= mid) & valid, 1, 0)

            accv = lax.fori_loop(0, nv, cc, jnp.zeros((_L,), jnp.int32))
            ge = jnp.sum(accv) >= _K
            return (jnp.where(ge, mid, lo_), jnp.where(ge, hi_, mid))

        t, _ = lax.while_loop(bis_cond, bis_body, (lo, hi))

        @plsc.parallel_loop(0, _ROW_VREGS, 4, unroll=4)
        def mpass(j4):
            for u in range(4):
                j = j4 + u
                v = adjv[pl.ds(j * _L, _L)]
                bits = lax.bitcast_convert_type(v, jnp.int32)
                adjv[pl.ds(j * _L, _L)] = jnp.where(bits >= t, v, 0.0)

    issue_in(row0, 0)

    def pair_body(g, carry):
        r = g * 2
        issue_in(row0 + r + 1, 1)
        wait_in(0)

        @pl.when(g > 0)
        def _():
            wait_out(0)

        process_row(row0 + r, 0)
        issue_out(row0 + r, 0)

        @pl.when(r + 2 < _ROWS_PER_W)
        def _():
            issue_in(row0 + r + 2, 0)

        wait_in(1)

        @pl.when(g > 0)
        def _():
            wait_out(1)

        process_row(row0 + r + 1, 1)
        issue_out(row0 + r + 1, 1)
        return carry

    lax.fori_loop(0, _ROWS_PER_W // 2, pair_body, 0)
    wait_out(0)
    wait_out(1)


@jax.jit
def kernel(idx, W, A_prior, freeze_mask):
    del idx
    in_spec = pl.BlockSpec((_BLOCK_ROWS, _N), lambda i: (i, 0))
    tc_out = pl.pallas_call(
        _tc_body,
        grid=(_SPLIT // _BLOCK_ROWS,),
        in_specs=[in_spec, in_spec, in_spec],
        out_specs=pl.BlockSpec((_BLOCK_ROWS, _N), lambda i: (i, 0)),
        out_shape=jax.ShapeDtypeStruct((_N, _N), jnp.float32),
    )(W, A_prior, freeze_mask)

    mesh = plsc.VectorSubcoreMesh(core_axis_name="c", subcore_axis_name="s")
    sc_out = pl.kernel(
        _sc_body,
        out_type=jax.ShapeDtypeStruct((_SC_ROWS, _N), jnp.float32),
        mesh=mesh,
        compiler_params=pltpu.CompilerParams(needs_layout_passes=False),
        scratch_types=(
            [pltpu.VMEM((_N,), jnp.float32)] * 8
            + [pltpu.VMEM((_CAND,), jnp.int32)]
            + [pltpu.SemaphoreType.DMA] * 4
        ),
    )(W, A_prior, freeze_mask)
    return lax.dynamic_update_slice(tc_out, sc_out, (_SPLIT, 0))
